# baseline (speedup 1.0000x reference)
import functools

import jax
import jax.numpy as jnp
from jax import lax
from jax.experimental import pallas as pl
from jax.experimental.pallas import tpu as pltpu
from jax.experimental.pallas import tpu_sc as plsc

VOCAB = 100000
EMBED_DIM = 128
PACKED_WORDS = 4
BATCH = 4096
SEQ = 50
N = BATCH * SEQ

NUM_WORKERS = 32
B_PER_WORKER = BATCH // NUM_WORKERS
PER_WORKER = N // NUM_WORKERS
B_BLK = 8
BLK = B_BLK * SEQ
NBLK = B_PER_WORKER // B_BLK
CHUNK = 16
NCHUNK = BLK // CHUNK
SUBS = (128, 128, 128, 16)


def _sc_body(
    idx_hbm, qw_hbm, scl_hbm, out_hbm,
    idx_v, i4_v, sl_v, pw_v, out_v, gsem, osem,
):
    c = lax.axis_index("c")
    s = lax.axis_index("s")
    wid = s * 2 + c
    base = wid * PER_WORKER
    bbase = wid * B_PER_WORKER

    iota = lax.iota(jnp.int32, 16)
    masks = (jnp.int32(1) << iota, jnp.int32(1) << (iota + 16))

    def load_and_fire(b, p):
        off = base + b * BLK
        for sp, ln in enumerate(SUBS):
            pltpu.sync_copy(idx_hbm.at[pl.ds(off + sp * 128, ln)], idx_v[p][sp])

            def mk_idx(k, carry2, sp=sp, p=p):
                iv4 = idx_v[p][sp][pl.ds(k * CHUNK, CHUNK)] << 2
                for w in range(PACKED_WORDS):
                    i4_v[p][sp][w][pl.ds(k * CHUNK, CHUNK)] = iv4 + w if w else iv4
                return carry2

            lax.fori_loop(0, ln // CHUNK, mk_idx, 0)
        for sp, ln in enumerate(SUBS):
            for w in range(PACKED_WORDS):
                pltpu.async_copy(
                    qw_hbm.at[i4_v[p][sp][w]],
                    pw_v[p][w].at[pl.ds(sp * 128, ln)],
                    gsem[p],
                )
            pltpu.async_copy(
                scl_hbm.at[idx_v[p][sp]], sl_v[p].at[pl.ds(sp * 128, ln)], gsem[p]
            )

    def wait_gathers(p):
        for sp, ln in enumerate(SUBS):
            for w in range(PACKED_WORDS):
                pltpu.make_async_copy(
                    qw_hbm.at[i4_v[p][sp][w]],
                    pw_v[p][w].at[pl.ds(sp * 128, ln)],
                    gsem[p],
                ).wait()
            pltpu.make_async_copy(
                scl_hbm.at[idx_v[p][sp]], sl_v[p].at[pl.ds(sp * 128, ln)], gsem[p]
            ).wait()

    def fire_out(b, p):
        for r in range(B_BLK):
            pltpu.async_copy(
                out_v[p].at[pl.ds(r * SEQ, SEQ), :],
                out_hbm.at[bbase + b * B_BLK + r],
                osem[p],
            )

    def wait_out(b, p):
        for r in range(B_BLK):
            pltpu.make_async_copy(
                out_v[p].at[pl.ds(r * SEQ, SEQ), :],
                out_hbm.at[bbase + b * B_BLK + r],
                osem[p],
            ).wait()

    def compute(p):
        def chunk_body(k, carry2):
            svec = sl_v[p][pl.ds(k * CHUNK, CHUNK)]
            wvecs = [
                pw_v[p][w][pl.ds(k * CHUNK, CHUNK)] for w in range(PACKED_WORDS)
            ]
            t0 = k * CHUNK
            for j in range(CHUNK):
                t = t0 + j
                cj = jnp.full((16,), j, jnp.int32)
                sb = svec.at[cj].get(mode="promise_in_bounds")
                nsb = -sb
                for w in range(PACKED_WORDS):
                    wv = wvecs[w].at[cj].get(mode="promise_in_bounds")
                    for h in range(2):
                        m = masks[h]
                        val = jnp.where((wv & m) == m, sb, nsb)
                        out_v[p][t, pl.ds(w * 32 + h * 16, 16)] = val
            return carry2

        lax.fori_loop(0, NCHUNK, chunk_body, 0)

    load_and_fire(0, 0)

    def pair_body(g, carry):
        for p in range(2):
            b = g * 2 + p

            @pl.when(b + 1 < NBLK)
            def _(b=b, p=p):
                load_and_fire(b + 1, 1 - p)

            wait_gathers(p)

            @pl.when(b >= 2)
            def _(b=b, p=p):
                wait_out(b - 2, p)

            compute(p)
            fire_out(b, p)
        return carry

    lax.fori_loop(0, NBLK // 2, pair_body, 0)
    wait_out(NBLK - 2, (NBLK - 2) % 2)
    wait_out(NBLK - 1, (NBLK - 1) % 2)


TR_ROWS = 3328
TR_PER_W = TR_ROWS // NUM_WORKERS


def _tr_body(qw_hbm, out_hbm):
    c = lax.axis_index("c")
    s = lax.axis_index("s")
    wid = s * 2 + c
    r0 = wid * TR_PER_W
    qb = qw_hbm.bitcast(jnp.int32)
    pltpu.sync_copy(qb.at[pl.ds(r0, TR_PER_W)], out_hbm.at[pl.ds(r0, TR_PER_W)])


def kernel(input, qweight, embed_scale):
    B, L = input.shape
    V, P = qweight.shape
    flat_idx = input.reshape(-1)
    qbytes = jnp.pad(qweight.reshape(V * P), (0, TR_ROWS * 512 - V * P))
    qw2d = (
        qbytes.reshape(TR_ROWS, 128, 4)
        .transpose(0, 2, 1)
        .reshape(TR_ROWS * 4, 128)
    )
    scl_flat = embed_scale.reshape(V)

    mesh_tr = plsc.VectorSubcoreMesh(core_axis_name="c", subcore_axis_name="s")
    tr = functools.partial(
        pl.kernel,
        mesh=mesh_tr,
        compiler_params=pltpu.CompilerParams(needs_layout_passes=False),
        out_type=jax.ShapeDtypeStruct((TR_ROWS, 128), jnp.int32),
    )(_tr_body)
    qw_flat = tr(qw2d).reshape(TR_ROWS * 128)

    mesh = plsc.VectorSubcoreMesh(core_axis_name="c", subcore_axis_name="s")
    sub_idx = [pltpu.VMEM((ln,), jnp.int32) for ln in SUBS]
    sub_i4 = [[pltpu.VMEM((ln,), jnp.int32)] * PACKED_WORDS for ln in SUBS]
    run = functools.partial(
        pl.kernel,
        mesh=mesh,
        compiler_params=pltpu.CompilerParams(needs_layout_passes=False),
        out_type=jax.ShapeDtypeStruct((BATCH, SEQ, EMBED_DIM), jnp.float32),
        scratch_types=[
            [sub_idx] * 2,
            [sub_i4] * 2,
            [pltpu.VMEM((BLK,), jnp.float32)] * 2,
            [[pltpu.VMEM((BLK,), jnp.int32)] * PACKED_WORDS] * 2,
            [pltpu.VMEM((BLK, EMBED_DIM), jnp.float32)] * 2,
            [pltpu.SemaphoreType.DMA] * 2,
            [pltpu.SemaphoreType.DMA] * 2,
        ],
    )(_sc_body)

    return run(flat_idx, qw_flat, scl_flat)


# speedup vs baseline: 1.5411x; 1.5411x over previous
import functools

import jax
import jax.numpy as jnp
from jax import lax
from jax.experimental import pallas as pl
from jax.experimental.pallas import tpu as pltpu
from jax.experimental.pallas import tpu_sc as plsc

VOCAB = 100000
EMBED_DIM = 128
PACKED_WORDS = 4
BATCH = 4096
SEQ = 50
N = BATCH * SEQ

NUM_WORKERS = 32
B_PER_WORKER = BATCH // NUM_WORKERS
PER_WORKER = N // NUM_WORKERS
B_BLK = 8
BLK = B_BLK * SEQ
NBLK = B_PER_WORKER // B_BLK
CHUNK = 16
NCHUNK = BLK // CHUNK
SUBS = (128, 128, 128, 16)


def _sc_body(
    idx_hbm, qw_hbm, scl_hbm, out_hbm,
    idx_v, i4_v, sl_v, pw_v, out_v, gsem, osem,
):
    c = lax.axis_index("c")
    s = lax.axis_index("s")
    wid = s * 2 + c
    base = wid * PER_WORKER
    bbase = wid * B_PER_WORKER

    iota = lax.iota(jnp.int32, 16)
    masks = (jnp.int32(1) << iota, jnp.int32(1) << (iota + 16))

    def load_and_fire(b, p):
        off = base + b * BLK
        for sp, ln in enumerate(SUBS):
            pltpu.sync_copy(idx_hbm.at[pl.ds(off + sp * 128, ln)], idx_v[p][sp])

            def mk_idx(k, carry2, sp=sp, p=p):
                iv4 = idx_v[p][sp][pl.ds(k * CHUNK, CHUNK)] << 2
                for w in range(PACKED_WORDS):
                    i4_v[p][sp][w][pl.ds(k * CHUNK, CHUNK)] = iv4 + w if w else iv4
                return carry2

            lax.fori_loop(0, ln // CHUNK, mk_idx, 0)
        for sp, ln in enumerate(SUBS):
            for w in range(PACKED_WORDS):
                pltpu.async_copy(
                    qw_hbm.at[i4_v[p][sp][w]],
                    pw_v[p][w].at[pl.ds(sp * 128, ln)],
                    gsem[p],
                )
            pltpu.async_copy(
                scl_hbm.at[idx_v[p][sp]], sl_v[p].at[pl.ds(sp * 128, ln)], gsem[p]
            )

    def wait_gathers(p):
        for sp, ln in enumerate(SUBS):
            for w in range(PACKED_WORDS):
                pltpu.make_async_copy(
                    qw_hbm.at[i4_v[p][sp][w]],
                    pw_v[p][w].at[pl.ds(sp * 128, ln)],
                    gsem[p],
                ).wait()
            pltpu.make_async_copy(
                scl_hbm.at[idx_v[p][sp]], sl_v[p].at[pl.ds(sp * 128, ln)], gsem[p]
            ).wait()

    def fire_out(b, p):
        for r in range(B_BLK):
            pltpu.async_copy(
                out_v[p].at[pl.ds(r * SEQ, SEQ), :],
                out_hbm.at[bbase + b * B_BLK + r],
                osem[p],
            )

    def wait_out(b, p):
        for r in range(B_BLK):
            pltpu.make_async_copy(
                out_v[p].at[pl.ds(r * SEQ, SEQ), :],
                out_hbm.at[bbase + b * B_BLK + r],
                osem[p],
            ).wait()

    def compute(p):
        def chunk_body(k, carry2):
            svec = sl_v[p][pl.ds(k * CHUNK, CHUNK)]
            wvecs = [
                pw_v[p][w][pl.ds(k * CHUNK, CHUNK)] for w in range(PACKED_WORDS)
            ]
            t0 = k * CHUNK
            for j in range(CHUNK):
                t = t0 + j
                cj = jnp.full((16,), j, jnp.int32)
                sb = svec.at[cj].get(mode="promise_in_bounds")
                nsb = -sb
                for w in range(PACKED_WORDS):
                    wv = wvecs[w].at[cj].get(mode="promise_in_bounds")
                    for h in range(2):
                        m = masks[h]
                        val = jnp.where((wv & m) == m, sb, nsb)
                        out_v[p][t, pl.ds(w * 32 + h * 16, 16)] = val
            return carry2

        lax.fori_loop(0, NCHUNK, chunk_body, 0)

    load_and_fire(0, 0)

    def pair_body(g, carry):
        for p in range(2):
            b = g * 2 + p

            @pl.when(b + 1 < NBLK)
            def _(b=b, p=p):
                load_and_fire(b + 1, 1 - p)

            wait_gathers(p)

            @pl.when(b >= 2)
            def _(b=b, p=p):
                wait_out(b - 2, p)

            compute(p)
            fire_out(b, p)
        return carry

    lax.fori_loop(0, NBLK // 2, pair_body, 0)
    wait_out(NBLK - 2, (NBLK - 2) % 2)
    wait_out(NBLK - 1, (NBLK - 1) % 2)


TR_ROWS = 3328
TR_PER_W = TR_ROWS // NUM_WORKERS


def _tr_body(qw_hbm, out_hbm):
    c = lax.axis_index("c")
    s = lax.axis_index("s")
    wid = s * 2 + c
    r0 = wid * TR_PER_W
    qb = qw_hbm.bitcast(jnp.int32)
    pltpu.sync_copy(qb.at[pl.ds(r0, TR_PER_W)], out_hbm.at[pl.ds(r0, TR_PER_W)])


def kernel(input, qweight, embed_scale):
    B, L = input.shape
    V, P = qweight.shape
    flat_idx = input.reshape(-1)
    qw_flat = jax.lax.bitcast_convert_type(
        qweight.reshape(V * P // 4, 4), jnp.int32
    )
    scl_flat = embed_scale.reshape(V)

    mesh = plsc.VectorSubcoreMesh(core_axis_name="c", subcore_axis_name="s")
    sub_idx = [pltpu.VMEM((ln,), jnp.int32) for ln in SUBS]
    sub_i4 = [[pltpu.VMEM((ln,), jnp.int32)] * PACKED_WORDS for ln in SUBS]
    run = functools.partial(
        pl.kernel,
        mesh=mesh,
        compiler_params=pltpu.CompilerParams(needs_layout_passes=False),
        out_type=jax.ShapeDtypeStruct((BATCH, SEQ, EMBED_DIM), jnp.float32),
        scratch_types=[
            [sub_idx] * 2,
            [sub_i4] * 2,
            [pltpu.VMEM((BLK,), jnp.float32)] * 2,
            [[pltpu.VMEM((BLK,), jnp.int32)] * PACKED_WORDS] * 2,
            [pltpu.VMEM((BLK, EMBED_DIM), jnp.float32)] * 2,
            [pltpu.SemaphoreType.DMA] * 2,
            [pltpu.SemaphoreType.DMA] * 2,
        ],
    )(_sc_body)

    return run(flat_idx, qw_flat, scl_flat)


# speedup vs baseline: 2.2229x; 1.4424x over previous
import functools

import jax
import jax.numpy as jnp
from jax import lax
from jax.experimental import pallas as pl
from jax.experimental.pallas import tpu as pltpu
from jax.experimental.pallas import tpu_sc as plsc

VOCAB = 100000
EMBED_DIM = 128
PACKED_WORDS = 4
BATCH = 4096
SEQ = 50
N = BATCH * SEQ

NUM_WORKERS = 32
B_PER_WORKER = BATCH // NUM_WORKERS
PER_WORKER = N // NUM_WORKERS
B_BLK = 8
BLK = B_BLK * SEQ
NBLK = B_PER_WORKER // B_BLK
CHUNK = 16
NCHUNK = BLK // CHUNK
SUBS = (128, 128, 128, 16)


def _sc_body(
    idx_hbm, qw_hbm, scl_hbm, out_hbm,
    idx_v, i4_v, sl_v, pw_v, out_v, gsem, osem,
):
    c = lax.axis_index("c")
    s = lax.axis_index("s")
    wid = s * 2 + c
    base = wid * PER_WORKER
    bbase = wid * B_PER_WORKER

    iota = lax.iota(jnp.int32, 16)
    masks = (jnp.int32(1) << iota, jnp.int32(1) << (iota + 16))

    def load_and_fire(b, p):
        off = base + b * BLK
        for sp, ln in enumerate(SUBS):
            pltpu.sync_copy(idx_hbm.at[pl.ds(off + sp * 128, ln)], idx_v[p][sp])

            def mk_idx(k, carry2, sp=sp, p=p):
                iv4 = idx_v[p][sp][pl.ds(k * CHUNK, CHUNK)] << 2
                for w in range(PACKED_WORDS):
                    i4_v[p][sp][w][pl.ds(k * CHUNK, CHUNK)] = iv4 + w if w else iv4
                return carry2

            lax.fori_loop(0, ln // CHUNK, mk_idx, 0)
        for sp, ln in enumerate(SUBS):
            for w in range(PACKED_WORDS):
                pltpu.async_copy(
                    qw_hbm.at[i4_v[p][sp][w]],
                    pw_v[p][w].at[pl.ds(sp * 128, ln)],
                    gsem[p],
                )
            pltpu.async_copy(
                scl_hbm.at[idx_v[p][sp]], sl_v[p].at[pl.ds(sp * 128, ln)], gsem[p]
            )

    def wait_gathers(p):
        for sp, ln in enumerate(SUBS):
            for w in range(PACKED_WORDS):
                pltpu.make_async_copy(
                    qw_hbm.at[i4_v[p][sp][w]],
                    pw_v[p][w].at[pl.ds(sp * 128, ln)],
                    gsem[p],
                ).wait()
            pltpu.make_async_copy(
                scl_hbm.at[idx_v[p][sp]], sl_v[p].at[pl.ds(sp * 128, ln)], gsem[p]
            ).wait()

    def fire_out(b, p):
        for r in range(B_BLK):
            pltpu.async_copy(
                out_v[p].at[pl.ds(r * SEQ, SEQ), :],
                out_hbm.at[bbase + b * B_BLK + r],
                osem[p],
            )

    def wait_out(b, p):
        for r in range(B_BLK):
            pltpu.make_async_copy(
                out_v[p].at[pl.ds(r * SEQ, SEQ), :],
                out_hbm.at[bbase + b * B_BLK + r],
                osem[p],
            ).wait()

    def compute(p):
        def chunk_body(k, carry2):
            svec = sl_v[p][pl.ds(k * CHUNK, CHUNK)]
            wvecs = [
                pw_v[p][w][pl.ds(k * CHUNK, CHUNK)] for w in range(PACKED_WORDS)
            ]
            t0 = k * CHUNK
            for j in range(CHUNK):
                t = t0 + j
                cj = jnp.full((16,), j, jnp.int32)
                sb = svec.at[cj].get(mode="promise_in_bounds")
                nsb = -sb
                for w in range(PACKED_WORDS):
                    wv = wvecs[w].at[cj].get(mode="promise_in_bounds")
                    for h in range(2):
                        m = masks[h]
                        val = jnp.where((wv & m) == m, sb, nsb)
                        out_v[p][t, pl.ds(w * 32 + h * 16, 16)] = val
            return carry2

        lax.fori_loop(0, NCHUNK, chunk_body, 0)

    load_and_fire(0, 0)

    def pair_body(g, carry):
        for p in range(2):
            b = g * 2 + p

            @pl.when(b + 1 < NBLK)
            def _(b=b, p=p):
                load_and_fire(b + 1, 1 - p)

            wait_gathers(p)

            @pl.when(b >= 2)
            def _(b=b, p=p):
                wait_out(b - 2, p)

            compute(p)
            fire_out(b, p)
        return carry

    lax.fori_loop(0, NBLK // 2, pair_body, 0)
    wait_out(NBLK - 2, (NBLK - 2) % 2)
    wait_out(NBLK - 1, (NBLK - 1) % 2)


TR_ROWS = 3328
TR_PER_W = TR_ROWS // NUM_WORKERS


def _tr_body(qw_hbm, out_hbm):
    c = lax.axis_index("c")
    s = lax.axis_index("s")
    wid = s * 2 + c
    r0 = wid * TR_PER_W
    qb = qw_hbm.bitcast(jnp.int32)
    pltpu.sync_copy(qb.at[pl.ds(r0, TR_PER_W)], out_hbm.at[pl.ds(r0, TR_PER_W)])


def kernel(input, qweight, embed_scale):
    B, L = input.shape
    V, P = qweight.shape
    flat_idx = input.reshape(-1)
    qw_flat = jax.lax.bitcast_convert_type(
        qweight.reshape(V, P // 4, 4), jnp.int32
    ).reshape(V * PACKED_WORDS)
    scl_flat = embed_scale.reshape(V)

    mesh = plsc.VectorSubcoreMesh(core_axis_name="c", subcore_axis_name="s")
    sub_idx = [pltpu.VMEM((ln,), jnp.int32) for ln in SUBS]
    sub_i4 = [[pltpu.VMEM((ln,), jnp.int32)] * PACKED_WORDS for ln in SUBS]
    run = functools.partial(
        pl.kernel,
        mesh=mesh,
        compiler_params=pltpu.CompilerParams(needs_layout_passes=False),
        out_type=jax.ShapeDtypeStruct((BATCH, SEQ, EMBED_DIM), jnp.float32),
        scratch_types=[
            [sub_idx] * 2,
            [sub_i4] * 2,
            [pltpu.VMEM((BLK,), jnp.float32)] * 2,
            [[pltpu.VMEM((BLK,), jnp.int32)] * PACKED_WORDS] * 2,
            [pltpu.VMEM((BLK, EMBED_DIM), jnp.float32)] * 2,
            [pltpu.SemaphoreType.DMA] * 2,
            [pltpu.SemaphoreType.DMA] * 2,
        ],
    )(_sc_body)

    return run(flat_idx, qw_flat, scl_flat)


# speedup vs baseline: 2.8976x; 1.3035x over previous
import functools

import jax
import jax.numpy as jnp
from jax import lax
from jax.experimental import pallas as pl
from jax.experimental.pallas import tpu as pltpu
from jax.experimental.pallas import tpu_sc as plsc

VOCAB = 100000
EMBED_DIM = 128
PACKED_WORDS = 4
BATCH = 4096
SEQ = 50
N = BATCH * SEQ

NUM_WORKERS = 32
B_PER_WORKER = BATCH // NUM_WORKERS
PER_WORKER = N // NUM_WORKERS
B_BLK = 8
BLK = B_BLK * SEQ
NBLK = B_PER_WORKER // B_BLK
CHUNK = 16
NCHUNK = BLK // CHUNK
SUBS = (128, 128, 128, 16)


def _sc_body(
    idx_hbm, qw0, qw1, qw2, qw3, scl_hbm, out_hbm,
    idx_v, sl_v, pw_v, out_v, gsem, osem,
):
    c = lax.axis_index("c")
    s = lax.axis_index("s")
    wid = s * 2 + c
    base = wid * PER_WORKER
    bbase = wid * B_PER_WORKER
    qw_refs = (qw0, qw1, qw2, qw3)

    iota = lax.iota(jnp.int32, 16)
    masks = (jnp.int32(1) << iota, jnp.int32(1) << (iota + 16))

    def load_and_fire(b, p):
        off = base + b * BLK
        for sp, ln in enumerate(SUBS):
            pltpu.sync_copy(idx_hbm.at[pl.ds(off + sp * 128, ln)], idx_v[p][sp])
        for sp, ln in enumerate(SUBS):
            for w in range(PACKED_WORDS):
                pltpu.async_copy(
                    qw_refs[w].at[idx_v[p][sp]],
                    pw_v[p][w].at[pl.ds(sp * 128, ln)],
                    gsem[p],
                )
            pltpu.async_copy(
                scl_hbm.at[idx_v[p][sp]], sl_v[p].at[pl.ds(sp * 128, ln)], gsem[p]
            )

    def wait_gathers(p):
        for sp, ln in enumerate(SUBS):
            for w in range(PACKED_WORDS):
                pltpu.make_async_copy(
                    qw_refs[w].at[idx_v[p][sp]],
                    pw_v[p][w].at[pl.ds(sp * 128, ln)],
                    gsem[p],
                ).wait()
            pltpu.make_async_copy(
                scl_hbm.at[idx_v[p][sp]], sl_v[p].at[pl.ds(sp * 128, ln)], gsem[p]
            ).wait()

    def fire_out(b, p):
        for r in range(B_BLK):
            pltpu.async_copy(
                out_v[p].at[pl.ds(r * SEQ, SEQ), :],
                out_hbm.at[bbase + b * B_BLK + r],
                osem[p],
            )

    def wait_out(b, p):
        for r in range(B_BLK):
            pltpu.make_async_copy(
                out_v[p].at[pl.ds(r * SEQ, SEQ), :],
                out_hbm.at[bbase + b * B_BLK + r],
                osem[p],
            ).wait()

    def compute(p):
        def chunk_body(k, carry2):
            svec = sl_v[p][pl.ds(k * CHUNK, CHUNK)]
            wvecs = [
                pw_v[p][w][pl.ds(k * CHUNK, CHUNK)] for w in range(PACKED_WORDS)
            ]
            t0 = k * CHUNK
            for j in range(CHUNK):
                t = t0 + j
                cj = jnp.full((16,), j, jnp.int32)
                sb = svec.at[cj].get(mode="promise_in_bounds")
                nsb = -sb
                for w in range(PACKED_WORDS):
                    wv = wvecs[w].at[cj].get(mode="promise_in_bounds")
                    for h in range(2):
                        m = masks[h]
                        val = jnp.where((wv & m) == m, sb, nsb)
                        out_v[p][t, pl.ds(w * 32 + h * 16, 16)] = val
            return carry2

        lax.fori_loop(0, NCHUNK, chunk_body, 0)

    load_and_fire(0, 0)

    def pair_body(g, carry):
        for p in range(2):
            b = g * 2 + p

            @pl.when(b + 1 < NBLK)
            def _(b=b, p=p):
                load_and_fire(b + 1, 1 - p)

            wait_gathers(p)

            @pl.when(b >= 2)
            def _(b=b, p=p):
                wait_out(b - 2, p)

            compute(p)
            fire_out(b, p)
        return carry

    lax.fori_loop(0, NBLK // 2, pair_body, 0)
    wait_out(NBLK - 2, (NBLK - 2) % 2)
    wait_out(NBLK - 1, (NBLK - 1) % 2)


def kernel(input, qweight, embed_scale):
    B, L = input.shape
    V, P = qweight.shape
    flat_idx = input.reshape(-1)
    qw_i32 = jax.lax.bitcast_convert_type(
        qweight.reshape(V, P // 4, 4), jnp.int32
    )
    qw_cols = tuple(qw_i32[:, w] for w in range(PACKED_WORDS))
    scl_flat = embed_scale.reshape(V)

    mesh = plsc.VectorSubcoreMesh(core_axis_name="c", subcore_axis_name="s")
    sub_idx = [pltpu.VMEM((ln,), jnp.int32) for ln in SUBS]
    run = functools.partial(
        pl.kernel,
        mesh=mesh,
        compiler_params=pltpu.CompilerParams(needs_layout_passes=False),
        out_type=jax.ShapeDtypeStruct((BATCH, SEQ, EMBED_DIM), jnp.float32),
        scratch_types=[
            [sub_idx] * 2,
            [pltpu.VMEM((BLK,), jnp.float32)] * 2,
            [[pltpu.VMEM((BLK,), jnp.int32)] * PACKED_WORDS] * 2,
            [pltpu.VMEM((BLK, EMBED_DIM), jnp.float32)] * 2,
            [pltpu.SemaphoreType.DMA] * 2,
            [pltpu.SemaphoreType.DMA] * 2,
        ],
    )(_sc_body)

    return run(flat_idx, *qw_cols, scl_flat)
